# baseline (speedup 1.0000x reference)
import functools

import jax
import jax.numpy as jnp
from jax import lax
from jax.experimental import pallas as pl
from jax.experimental.pallas import tpu as pltpu
from jax.experimental.pallas import tpu_sc as plsc

N_NEI = 4
D = 2048
NV = 20
B = 32
M = B * NV
BN = 512
PNV = 32
BIG = 3.0e38

_NC, _NS = 2, 16
NW = _NC * _NS


def _prep_body(x_ref, adj_ref):
    for b in range(B):
        xb = x_ref[b]
        g = lax.dot_general(xb, xb, (((1,), (1,)), ((), ())),
                            preferred_element_type=jnp.float32)
        sq = jnp.sum(xb * xb, axis=1)
        inner = -2.0 * g
        adj = (sq[None, :] + inner) + sq[:, None]
        a = jnp.concatenate(
            [adj, jnp.full((NV, PNV - NV), BIG, jnp.float32)], axis=1)
        a = jnp.concatenate(
            [a, jnp.full((PNV - NV, PNV), BIG, jnp.float32)], axis=0)
        adj_ref[b] = a.T


def _sc_select_body(adjT_hbm, sidx_hbm, adj_v, sidx_v):
    wid = lax.axis_index("s") * _NC + lax.axis_index("c")
    pltpu.sync_copy(adjT_hbm.at[wid], adj_v)
    iota = lax.broadcasted_iota(jnp.int32, (16,), 0)
    bigv = jnp.full((16,), BIG, jnp.float32)
    for c in range(2):
        taken = []
        for k in range(N_NEI):
            best = bigv
            bidx = jnp.zeros((16,), jnp.int32)
            for m in range(PNV):
                vm = adj_v[m, 16 * c:16 * (c + 1)]
                lt = vm < best
                for t in taken:
                    lt = lt & (t != m)
                best = jnp.where(lt, vm, best)
                bidx = jnp.where(lt, m, bidx)
            sidx_v[k, 16 * c:16 * (c + 1)] = bidx
            taken.append(bidx)
    pltpu.sync_copy(sidx_v, sidx_hbm.at[wid])


def _make_sc_select():
    return pl.kernel(
        _sc_select_body,
        out_type=jax.ShapeDtypeStruct((NW, N_NEI, PNV), jnp.int32),
        mesh=plsc.VectorSubcoreMesh(core_axis_name="c",
                                    subcore_axis_name="s"),
        scratch_types=[pltpu.VMEM((PNV, PNV), jnp.float32),
                       pltpu.VMEM((N_NEI, PNV), jnp.int32)],
    )


def _fused_body(x_ref, w_ref, sidx_ref, b_ref, out_ref, xbf_ref):
    j = pl.program_id(0)

    @pl.when(j == 0)
    def _cast():
        xbf_ref[...] = x_ref[...].reshape(M, D).astype(jnp.bfloat16)

    xbf = xbf_ref[...]
    wbf = w_ref[...].astype(jnp.bfloat16)
    dn = (((1,), (1,)), ((), ()))
    c1 = lax.dot_general(xbf, wbf[:, :D], dn,
                         preferred_element_type=jnp.float32)
    c2 = lax.dot_general(xbf, wbf[:, D:], dn,
                         preferred_element_type=jnp.float32)
    d = c1 - c2 + b_ref[...]

    iota_m = lax.broadcasted_iota(jnp.int32, (NV, NV), 1).astype(jnp.float32)
    for b in range(B):
        rows = slice(b * NV, (b + 1) * NV)
        c2b = c2[rows]
        idx_t = jnp.transpose(sidx_ref[b].astype(jnp.float32))
        p = None
        for k in range(N_NEI):
            oh = (idx_t[:NV, k][:, None] == iota_m).astype(jnp.float32)
            pk = lax.dot_general(oh, c2b, (((1,), (0,)), ((), ())),
                                 preferred_element_type=jnp.float32)
            p = pk if p is None else jnp.maximum(p, pk)
        h = jnp.maximum(d[rows] + p, 0.0)
        out_ref[b, :] = jnp.max(h, axis=0)


@functools.partial(jax.jit, static_argnames=())
def kernel(x, W, b):
    b2d = b.reshape(1, D)

    adjT = pl.pallas_call(
        _prep_body,
        grid=(1,),
        in_specs=[pl.BlockSpec((B, NV, D), lambda i: (0, 0, 0))],
        out_specs=pl.BlockSpec((NW, PNV, PNV), lambda i: (0, 0, 0)),
        out_shape=jax.ShapeDtypeStruct((NW, PNV, PNV), jnp.float32),
    )(x)

    sidx = _make_sc_select()(adjT)

    out = pl.pallas_call(
        _fused_body,
        grid=(D // BN,),
        in_specs=[
            pl.BlockSpec((B, NV, D), lambda j: (0, 0, 0)),
            pl.BlockSpec((BN, 2 * D), lambda j: (j, 0)),
            pl.BlockSpec((NW, N_NEI, PNV), lambda j: (0, 0, 0)),
            pl.BlockSpec((1, BN), lambda j: (0, j)),
        ],
        out_specs=pl.BlockSpec((B, BN), lambda j: (0, j)),
        out_shape=jax.ShapeDtypeStruct((B, D), jnp.float32),
        scratch_shapes=[pltpu.VMEM((M, D), jnp.bfloat16)],
    )(x, W, sidx, b2d)

    return out.reshape(B, D, 1, 1)
